# Initial kernel scaffold; baseline (speedup 1.0000x reference)
import jax
import jax.numpy as jnp
from jax.experimental import pallas as pl


def kernel(x, edge_index, W_nbr, W_root, b, gamma, beta):
    raise NotImplementedError("write your pallas kernel here")


# speedup vs baseline: 7.5020x; 7.5020x over previous
import functools

import jax
import jax.numpy as jnp
from jax import lax
from jax.experimental import pallas as pl
from jax.experimental.pallas import tpu as pltpu
from jax.experimental.pallas import tpu_sc as plsc

NC = 2
NS = 16
NW = NC * NS


def _ln_relu_body(x_ref, g_ref, b_ref, o_ref):
    xv = x_ref[...]
    mu = jnp.mean(xv, axis=1, keepdims=True)
    var = jnp.mean((xv - mu) ** 2, axis=1, keepdims=True)
    h = (xv - mu) * lax.rsqrt(var + 1e-5) * g_ref[...] + b_ref[...]
    o_ref[...] = jnp.maximum(h, 0.0)


def _ln_relu(x, gamma, beta, blk):
    n, d = x.shape
    return pl.pallas_call(
        _ln_relu_body,
        grid=(n // blk,),
        in_specs=[
            pl.BlockSpec((blk, d), lambda i: (i, 0)),
            pl.BlockSpec((1, d), lambda i: (0, 0)),
            pl.BlockSpec((1, d), lambda i: (0, 0)),
        ],
        out_specs=pl.BlockSpec((blk, d), lambda i: (i, 0)),
        out_shape=jax.ShapeDtypeStruct((n, d), jnp.float32),
    )(x, gamma.reshape(1, d), beta.reshape(1, d))


def _combine_body(x_ref, h_ref, p_ref, wn_ref, wr_ref, b_ref, o_ref):
    agg = p_ref[0] + p_ref[1]
    o_ref[...] = (
        x_ref[...]
        + b_ref[...]
        + jnp.dot(agg, wn_ref[...], preferred_element_type=jnp.float32)
        + jnp.dot(h_ref[...], wr_ref[...], preferred_element_type=jnp.float32)
    )


def _combine(x, h, parts, w_nbr, w_root, bias, blk):
    n, d = x.shape
    return pl.pallas_call(
        _combine_body,
        grid=(n // blk,),
        in_specs=[
            pl.BlockSpec((blk, d), lambda i: (i, 0)),
            pl.BlockSpec((blk, d), lambda i: (i, 0)),
            pl.BlockSpec((2, blk, d), lambda i: (0, i, 0)),
            pl.BlockSpec((d, d), lambda i: (0, 0)),
            pl.BlockSpec((d, d), lambda i: (0, 0)),
            pl.BlockSpec((1, d), lambda i: (0, 0)),
        ],
        out_specs=pl.BlockSpec((blk, d), lambda i: (i, 0)),
        out_shape=jax.ShapeDtypeStruct((n, d), jnp.float32),
    )(x, h, parts, w_nbr, w_root, bias.reshape(1, d))


def _sc_segment_sum(h, src, dst, zeros, n_pad, d, nb, bsz):
    rpt = n_pad // NS
    mesh = plsc.VectorSubcoreMesh(core_axis_name="c", subcore_axis_name="s")

    @functools.partial(
        pl.kernel,
        mesh=mesh,
        out_type=jax.ShapeDtypeStruct((NC, n_pad, d), jnp.float32),
        scratch_types=[
            pltpu.VMEM((nb, bsz), jnp.int32),
            pltpu.VMEM((nb, bsz), jnp.int32),
            pltpu.VMEM((bsz, d), jnp.float32),
            pltpu.VMEM_SHARED((n_pad, d), jnp.float32),
            pltpu.SemaphoreType.DMA,
        ],
    )
    def k(h_hbm, src_hbm, dst_hbm, zeros_hbm, out_hbm, src_v, dst_v, rows_v, acc_s, sem):
        cid = lax.axis_index("c")
        sid = lax.axis_index("s")
        wid = cid * NS + sid

        pltpu.sync_copy(zeros_hbm.at[pl.ds(sid * rpt, rpt)],
                        acc_s.at[pl.ds(sid * rpt, rpt)])
        pltpu.sync_copy(src_hbm.at[wid], src_v)
        pltpu.sync_copy(dst_hbm.at[wid], dst_v)
        plsc.subcore_barrier()

        def body(j, carry):
            pltpu.async_copy(h_hbm.at[src_v.at[j]], rows_v, sem).wait()
            pltpu.sync_copy(rows_v, acc_s.at[dst_v.at[j]], add=True)
            return carry

        lax.fori_loop(0, nb, body, 0)

        plsc.subcore_barrier()
        pltpu.sync_copy(acc_s.at[pl.ds(sid * rpt, rpt)],
                        out_hbm.at[cid, pl.ds(sid * rpt, rpt)])

    return k(h, src, dst, zeros)


def kernel(x, edge_index, W_nbr, W_root, b, gamma, beta):
    n, d = x.shape
    e = edge_index.shape[1]
    ept = e // NW
    bsz = 80
    nb = ept // bsz

    n_pad = ((n + 8 * NS - 1) // (8 * NS)) * (8 * NS)
    src = edge_index[0].reshape(NW, nb, bsz)
    dst = edge_index[1].reshape(NW, nb, bsz)
    zeros = jnp.zeros((n_pad, d), jnp.float32)

    h = _ln_relu(x, gamma, beta, blk=2000)
    parts = _sc_segment_sum(h, src, dst, zeros, n_pad, d, nb, bsz)
    return _combine(x, h, parts, W_nbr, W_root, b, blk=2000)


# speedup vs baseline: 9.0293x; 1.2036x over previous
import functools

import jax
import jax.numpy as jnp
from jax import lax
from jax.experimental import pallas as pl
from jax.experimental.pallas import tpu as pltpu
from jax.experimental.pallas import tpu_sc as plsc

NC = 2
NS = 16
NW = NC * NS


def _ln_relu_body(x_ref, g_ref, b_ref, o_ref):
    xv = x_ref[...]
    mu = jnp.mean(xv, axis=1, keepdims=True)
    var = jnp.mean((xv - mu) ** 2, axis=1, keepdims=True)
    h = (xv - mu) * lax.rsqrt(var + 1e-5) * g_ref[...] + b_ref[...]
    o_ref[...] = jnp.maximum(h, 0.0)


def _ln_relu(x, gamma, beta, blk):
    n, d = x.shape
    return pl.pallas_call(
        _ln_relu_body,
        grid=(n // blk,),
        in_specs=[
            pl.BlockSpec((blk, d), lambda i: (i, 0)),
            pl.BlockSpec((1, d), lambda i: (0, 0)),
            pl.BlockSpec((1, d), lambda i: (0, 0)),
        ],
        out_specs=pl.BlockSpec((blk, d), lambda i: (i, 0)),
        out_shape=jax.ShapeDtypeStruct((n, d), jnp.float32),
    )(x, gamma.reshape(1, d), beta.reshape(1, d))


def _combine_body(x_ref, h_ref, p_ref, wn_ref, wr_ref, b_ref, o_ref):
    agg = p_ref[0] + p_ref[1]
    o_ref[...] = (
        x_ref[...]
        + b_ref[...]
        + jnp.dot(agg, wn_ref[...], preferred_element_type=jnp.float32)
        + jnp.dot(h_ref[...], wr_ref[...], preferred_element_type=jnp.float32)
    )


def _combine(x, h, parts, w_nbr, w_root, bias, blk):
    n, d = x.shape
    return pl.pallas_call(
        _combine_body,
        grid=(n // blk,),
        in_specs=[
            pl.BlockSpec((blk, d), lambda i: (i, 0)),
            pl.BlockSpec((blk, d), lambda i: (i, 0)),
            pl.BlockSpec((2, blk, d), lambda i: (0, i, 0)),
            pl.BlockSpec((d, d), lambda i: (0, 0)),
            pl.BlockSpec((d, d), lambda i: (0, 0)),
            pl.BlockSpec((1, d), lambda i: (0, 0)),
        ],
        out_specs=pl.BlockSpec((blk, d), lambda i: (i, 0)),
        out_shape=jax.ShapeDtypeStruct((n, d), jnp.float32),
    )(x, h, parts, w_nbr, w_root, bias.reshape(1, d))


def _sc_segment_sum(h, src, dst, zeros, n_pad, d, ng, gsz, bsz):
    rpt = n_pad // NS
    mesh = plsc.VectorSubcoreMesh(core_axis_name="c", subcore_axis_name="s")

    @functools.partial(
        pl.kernel,
        mesh=mesh,
        out_type=jax.ShapeDtypeStruct((NC, n_pad, d), jnp.float32),
        scratch_types=[
            pltpu.VMEM((gsz, bsz), jnp.int32),
            pltpu.VMEM((gsz, bsz), jnp.int32),
            pltpu.VMEM((2, bsz, d), jnp.float32),
            pltpu.VMEM_SHARED((n_pad, d), jnp.float32),
            pltpu.SemaphoreType.DMA,
            pltpu.SemaphoreType.DMA,
        ],
    )
    def k(h_hbm, src_hbm, dst_hbm, zeros_hbm, out_hbm, src_v, dst_v, rows_v, acc_s, sem0, sem1):
        cid = lax.axis_index("c")
        sid = lax.axis_index("s")
        wid = cid * NS + sid

        pltpu.sync_copy(zeros_hbm.at[pl.ds(sid * rpt, rpt)],
                        acc_s.at[pl.ds(sid * rpt, rpt)])
        plsc.subcore_barrier()

        for g in range(ng):
            pltpu.sync_copy(src_hbm.at[wid, g], src_v)
            pltpu.sync_copy(dst_hbm.at[wid, g], dst_v)
            pltpu.async_copy(h_hbm.at[src_v.at[0]], rows_v.at[0], sem0)

            def body(p, carry):
                b0 = 2 * p
                pltpu.make_async_copy(h_hbm.at[src_v.at[b0]], rows_v.at[0], sem0).wait()
                pltpu.async_copy(h_hbm.at[src_v.at[b0 + 1]], rows_v.at[1], sem1)
                pltpu.sync_copy(rows_v.at[0], acc_s.at[dst_v.at[b0]], add=True)
                pltpu.make_async_copy(h_hbm.at[src_v.at[b0 + 1]], rows_v.at[1], sem1).wait()
                pltpu.async_copy(h_hbm.at[src_v.at[b0 + 2]], rows_v.at[0], sem0)
                pltpu.sync_copy(rows_v.at[1], acc_s.at[dst_v.at[b0 + 1]], add=True)
                return carry

            lax.fori_loop(0, (gsz - 1) // 2, body, 0)
            pltpu.make_async_copy(h_hbm.at[src_v.at[gsz - 1]], rows_v.at[0], sem0).wait()
            pltpu.sync_copy(rows_v.at[0], acc_s.at[dst_v.at[gsz - 1]], add=True)

        plsc.subcore_barrier()
        pltpu.sync_copy(acc_s.at[pl.ds(sid * rpt, rpt)],
                        out_hbm.at[cid, pl.ds(sid * rpt, rpt)])

    return k(h, src, dst, zeros)


def kernel(x, edge_index, W_nbr, W_root, b, gamma, beta):
    n, d = x.shape
    e = edge_index.shape[1]
    ept = e // NW
    bsz = 80
    gsz = 25
    ng = ept // (bsz * gsz)

    n_pad = ((n + 8 * NS - 1) // (8 * NS)) * (8 * NS)
    src = edge_index[0].reshape(NW, ng, gsz, bsz)
    dst = edge_index[1].reshape(NW, ng, gsz, bsz)
    zeros = jnp.zeros((n_pad, d), jnp.float32)

    h = _ln_relu(x, gamma, beta, blk=2000)
    parts = _sc_segment_sum(h, src, dst, zeros, n_pad, d, ng, gsz, bsz)
    return _combine(x, h, parts, W_nbr, W_root, b, blk=2000)
